# Initial kernel scaffold; baseline (speedup 1.0000x reference)
import jax
import jax.numpy as jnp
from jax.experimental import pallas as pl


def kernel(adj, tokens, emb_table, W1, b1, W2, b2, W3, b3):
    raise NotImplementedError("write your pallas kernel here")


# speedup vs baseline: 1.5189x; 1.5189x over previous
import jax
import jax.numpy as jnp
from jax.experimental import pallas as pl

B, N, D, VOCAB = 8, 2048, 16, 30
VP = 32


def _gcn_body(adj_ref, tok_ref, emb_ref, w_ref, b_ref, out_ref):
    A = adj_ref[0].astype(jnp.bfloat16)
    onehot = (tok_ref[0] == jax.lax.broadcasted_iota(jnp.int32, (N, VP), 1))
    h = jnp.dot(onehot.astype(jnp.bfloat16), emb_ref[...].astype(jnp.bfloat16),
                preferred_element_type=jnp.float32)
    sums = []
    for l in range(3):
        g = jnp.dot(A, h.astype(jnp.bfloat16),
                    preferred_element_type=jnp.float32)
        z = jnp.dot(g.astype(jnp.bfloat16), w_ref[l].astype(jnp.bfloat16),
                    preferred_element_type=jnp.float32) + b_ref[l]
        h = jnp.maximum(z, 0.0)
        sums.append(jnp.sum(h, axis=0, keepdims=True))
    out_ref[0] = jnp.concatenate(sums, axis=0)


def kernel(adj, tokens, emb_table, W1, b1, W2, b2, W3, b3):
    tok = tokens.astype(jnp.int32).reshape(B, N, 1)
    emb_pad = jnp.zeros((VP, D), jnp.float32).at[:VOCAB].set(emb_table)
    Ws = jnp.stack([W1, W2, W3])
    bs = jnp.stack([b1, b2, b3]).reshape(3, 1, D)
    out = pl.pallas_call(
        _gcn_body,
        grid=(B,),
        in_specs=[
            pl.BlockSpec((1, N, N), lambda b: (b, 0, 0)),
            pl.BlockSpec((1, N, 1), lambda b: (b, 0, 0)),
            pl.BlockSpec((VP, D), lambda b: (0, 0)),
            pl.BlockSpec((3, D, D), lambda b: (0, 0, 0)),
            pl.BlockSpec((3, 1, D), lambda b: (0, 0, 0)),
        ],
        out_specs=pl.BlockSpec((1, 3, D), lambda b: (b, 0, 0)),
        out_shape=jax.ShapeDtypeStruct((B, 3, D), jnp.float32),
    )(adj, tok, emb_pad, Ws, bs)
    return out.reshape(B, 3 * D)


# speedup vs baseline: 1.5502x; 1.0206x over previous
import jax
import jax.numpy as jnp
from jax.experimental import pallas as pl

B, N, D, VOCAB = 8, 2048, 16, 30
VP = 32


def _gcn_body(adj_ref, tok_ref, emb_ref, w_ref, b_ref, out_ref):
    A = adj_ref[0]
    onehot = (tok_ref[0] == jax.lax.broadcasted_iota(jnp.int32, (N, VP), 1))
    ew1 = jnp.dot(emb_ref[...], w_ref[0], preferred_element_type=jnp.float32)
    t = jnp.dot(onehot.astype(jnp.float32), ew1,
                preferred_element_type=jnp.float32)
    sums = []
    for l in range(3):
        z = jnp.dot(A, t, preferred_element_type=jnp.float32) + b_ref[l]
        h = jnp.maximum(z, 0.0)
        sums.append(jnp.sum(h, axis=0, keepdims=True))
        if l < 2:
            t = jnp.dot(h, w_ref[l + 1], preferred_element_type=jnp.float32)
    out_ref[0] = jnp.concatenate(sums, axis=0)


def kernel(adj, tokens, emb_table, W1, b1, W2, b2, W3, b3):
    tok = tokens.astype(jnp.int32).reshape(B, N, 1)
    emb_pad = jnp.zeros((VP, D), jnp.float32).at[:VOCAB].set(emb_table)
    Ws = jnp.stack([W1, W2, W3])
    bs = jnp.stack([b1, b2, b3]).reshape(3, 1, D)
    out = pl.pallas_call(
        _gcn_body,
        grid=(B,),
        in_specs=[
            pl.BlockSpec((1, N, N), lambda b: (b, 0, 0)),
            pl.BlockSpec((1, N, 1), lambda b: (b, 0, 0)),
            pl.BlockSpec((VP, D), lambda b: (0, 0)),
            pl.BlockSpec((3, D, D), lambda b: (0, 0, 0)),
            pl.BlockSpec((3, 1, D), lambda b: (0, 0, 0)),
        ],
        out_specs=pl.BlockSpec((1, 3, D), lambda b: (b, 0, 0)),
        out_shape=jax.ShapeDtypeStruct((B, 3, D), jnp.float32),
    )(adj, tok, emb_pad, Ws, bs)
    return out.reshape(B, 3 * D)


# speedup vs baseline: 1.6524x; 1.0659x over previous
import jax
import jax.numpy as jnp
from jax.experimental import pallas as pl

B, N, D, VOCAB = 8, 2048, 16, 30


def _gcn_body(adj_ref, tok_ref, emb_ref, w1_ref, b1_ref, w2_ref, b2_ref,
              w3_ref, b3_ref, out_ref):
    A = adj_ref[0]
    ot = (tok_ref[0] == jax.lax.broadcasted_iota(jnp.int32, (VOCAB, N), 0))
    ew1 = jnp.dot(emb_ref[...], w1_ref[...], preferred_element_type=jnp.float32)
    t = jax.lax.dot_general(ot.astype(jnp.float32), ew1,
                            (((0,), (0,)), ((), ())),
                            preferred_element_type=jnp.float32)
    sums = []
    for w_ref, b_ref in ((None, b1_ref), (w2_ref, b2_ref), (w3_ref, b3_ref)):
        if w_ref is not None:
            t = jnp.dot(t, w_ref[...], preferred_element_type=jnp.float32)
        z = jnp.dot(A, t, preferred_element_type=jnp.float32) + b_ref[...]
        t = jnp.maximum(z, 0.0)
        sums.append(jnp.sum(t, axis=0, keepdims=True))
    out_ref[0] = jnp.concatenate(sums, axis=0)


def kernel(adj, tokens, emb_table, W1, b1, W2, b2, W3, b3):
    tok = tokens.astype(jnp.int32).reshape(B, 1, N)
    full = lambda s: pl.BlockSpec(s, lambda b: tuple(0 for _ in s))
    out = pl.pallas_call(
        _gcn_body,
        grid=(B,),
        in_specs=[
            pl.BlockSpec((1, N, N), lambda b: (b, 0, 0)),
            pl.BlockSpec((1, 1, N), lambda b: (b, 0, 0)),
            full((VOCAB, D)),
            full((D, D)), full((1, D)),
            full((D, D)), full((1, D)),
            full((D, D)), full((1, D)),
        ],
        out_specs=pl.BlockSpec((1, 3, D), lambda b: (b, 0, 0)),
        out_shape=jax.ShapeDtypeStruct((B, 3, D), jnp.float32),
    )(adj, tok, emb_table,
      W1, b1.reshape(1, D), W2, b2.reshape(1, D), W3, b3.reshape(1, D))
    return out.reshape(B, 3 * D)


# speedup vs baseline: 2.4787x; 1.5001x over previous
import jax
import jax.numpy as jnp
from jax.experimental import pallas as pl

B, N, D, VOCAB = 8, 2048, 16, 30


def _gcn_body(adj_ref, tok_ref, emb_ref, w1_ref, b1_ref, w2_ref, b2_ref,
              w3_ref, b3_ref, out_ref):
    A = adj_ref[0]
    ot = (tok_ref[0] == jax.lax.broadcasted_iota(jnp.int32, (VOCAB, N), 0))
    ew1 = jnp.dot(emb_ref[...], w1_ref[...], preferred_element_type=jnp.float32)
    t = jax.lax.dot_general(ot.astype(jnp.float32), ew1,
                            (((0,), (0,)), ((), ())),
                            preferred_element_type=jnp.float32)
    H = N // 2
    sums = []
    for w_ref, b_ref in ((None, b1_ref), (w2_ref, b2_ref), (w3_ref, b3_ref)):
        g0 = jnp.dot(A[:H], t, preferred_element_type=jnp.float32)
        g1 = jnp.dot(A[H:], t, preferred_element_type=jnp.float32)
        g = jnp.concatenate([g0, g1], axis=0)
        if w_ref is not None:
            g = jnp.dot(g, w_ref[...], preferred_element_type=jnp.float32)
        t = jnp.maximum(g + b_ref[...], 0.0)
        sums.append(jnp.sum(t, axis=0, keepdims=True))
    out_ref[0] = jnp.concatenate(sums, axis=0)


def kernel(adj, tokens, emb_table, W1, b1, W2, b2, W3, b3):
    tok = tokens.astype(jnp.int32).reshape(B, 1, N)
    full = lambda s: pl.BlockSpec(s, lambda b: tuple(0 for _ in s))
    out = pl.pallas_call(
        _gcn_body,
        grid=(B,),
        in_specs=[
            pl.BlockSpec((1, N, N), lambda b: (b, 0, 0)),
            pl.BlockSpec((1, 1, N), lambda b: (b, 0, 0)),
            full((VOCAB, D)),
            full((D, D)), full((1, D)),
            full((D, D)), full((1, D)),
            full((D, D)), full((1, D)),
        ],
        out_specs=pl.BlockSpec((1, 3, D), lambda b: (b, 0, 0)),
        out_shape=jax.ShapeDtypeStruct((B, 3, D), jnp.float32),
    )(adj, tok, emb_table,
      W1, b1.reshape(1, D), W2, b2.reshape(1, D), W3, b3.reshape(1, D))
    return out.reshape(B, 3 * D)
